# baseline (speedup 1.0000x reference)
import functools

import jax
import jax.numpy as jnp
from jax import lax
from jax.experimental import pallas as pl
from jax.experimental.pallas import tpu as pltpu
from jax.experimental.pallas import tpu_sc as plsc

N_NODES = 10000
N_EDGES = 160000
D_IN = 256
D_HID = 256
D_OUT = 128

DH = 128
N_SUB = 16
EPT = N_EDGES // N_SUB
EB = 128
NCH = 80
EPT_PAD = NCH * EB
NST = 5
CPS = NCH // NST
ACC_ROWS = N_NODES + 8
DUMP = N_NODES
RA = 624
RB = N_NODES - 15 * RA

TR = 1000
NT = N_NODES // TR


def _copy_rows(src_ref, dst_ref, s):
  @pl.when(s < 15)
  def _():
    r0 = pl.multiple_of(s * RA, 8)
    pltpu.sync_copy(src_ref.at[pl.ds(r0, RA)], dst_ref.at[pl.ds(r0, RA)])

  @pl.when(s == 15)
  def _():
    pltpu.sync_copy(src_ref.at[pl.ds(15 * RA, RB)],
                    dst_ref.at[pl.ds(15 * RA, RB)])


def _sc_agg(x0, x1, src3, dst3):
  mesh = plsc.VectorSubcoreMesh(core_axis_name="c", subcore_axis_name="s")

  @functools.partial(
      pl.kernel,
      out_type=(
          jax.ShapeDtypeStruct((N_NODES, DH), jnp.float32),
          jax.ShapeDtypeStruct((N_NODES, DH), jnp.float32),
      ),
      mesh=mesh,
      scratch_types=[
          pltpu.VMEM_SHARED((ACC_ROWS, DH), jnp.float32),
          pltpu.VMEM((CPS, EB), jnp.int32),
          pltpu.VMEM((CPS, EB), jnp.int32),
          pltpu.VMEM((EB, DH), jnp.float32),
          pltpu.VMEM((EB, DH), jnp.float32),
          pltpu.SemaphoreType.DMA,
          pltpu.SemaphoreType.DMA,
          pltpu.SemaphoreType.DMA,
          pltpu.SemaphoreType.DMA,
      ],
  )
  def agg_kernel(x0_hbm, x1_hbm, src_hbm, dst_hbm, out0_hbm, out1_hbm,
                 acc, src_v, dst_v, rows_a, rows_b, sem_a, sem_b,
                 sem_sa, sem_sb):
    c = lax.axis_index("c")
    s = lax.axis_index("s")

    def do_half(x_hbm, out_hbm):
      _copy_rows(x_hbm, acc, s)
      plsc.subcore_barrier()

      @pl.loop(0, NST)
      def _(t):
        ci = pltpu.async_copy(src_hbm.at[s, t], src_v, sem_sa)
        cj = pltpu.async_copy(dst_hbm.at[s, t], dst_v, sem_sb)
        ci.wait()
        cj.wait()

        pltpu.async_copy(x_hbm.at[src_v.at[0]], rows_a, sem_a)

        @pl.loop(0, CPS, step=2)
        def _(j):
          pltpu.async_copy(x_hbm.at[src_v.at[j + 1]], rows_b, sem_b)
          pltpu.make_async_copy(x_hbm.at[src_v.at[j]], rows_a, sem_a).wait()
          pltpu.sync_copy(rows_a, acc.at[dst_v.at[j]], add=True)

          @pl.when(j + 2 < CPS)
          def _():
            pltpu.async_copy(x_hbm.at[src_v.at[j + 2]], rows_a, sem_a)

          pltpu.make_async_copy(x_hbm.at[src_v.at[j + 1]], rows_b, sem_b).wait()
          pltpu.sync_copy(rows_b, acc.at[dst_v.at[j + 1]], add=True)

      plsc.subcore_barrier()
      _copy_rows(acc, out_hbm, s)

    @pl.when(c == 0)
    def _():
      do_half(x0_hbm, out0_hbm)

    @pl.when(c == 1)
    def _():
      do_half(x1_hbm, out1_hbm)

  return agg_kernel(x0, x1, src3, dst3)


def _tc_mlp_bn_relu(h0, h1, W1a, W1b, b1, gamma, beta):
  def body(h0_ref, h1_ref, w1a_ref, w1b_ref, b1_ref, g_ref, bt_ref,
           o0_ref, o1_ref, acc_ref, ss_ref):
    p = pl.program_id(0)
    i = pl.program_id(1)
    y = (jnp.dot(h0_ref[...], w1a_ref[...], preferred_element_type=jnp.float32)
         + jnp.dot(h1_ref[...], w1b_ref[...], preferred_element_type=jnp.float32)
         + b1_ref[...])

    @pl.when(jnp.logical_and(p == 0, i == 0))
    def _():
      acc_ref[...] = jnp.zeros_like(acc_ref)

    @pl.when(p == 0)
    def _():
      acc_ref[0:1, :] += jnp.sum(y, axis=0, keepdims=True)
      acc_ref[1:2, :] += jnp.sum(y * y, axis=0, keepdims=True)

    @pl.when(p == 1)
    def _():
      @pl.when(i == 0)
      def _():
        mean = acc_ref[0:1, :] * (1.0 / N_NODES)
        var = acc_ref[1:2, :] * (1.0 / N_NODES) - mean * mean
        rstd = lax.rsqrt(var + 1e-5)
        ss_ref[0:1, :] = g_ref[...] * rstd
        ss_ref[1:2, :] = bt_ref[...] - g_ref[...] * rstd * mean

      h = jnp.maximum(y * ss_ref[0:1, :] + ss_ref[1:2, :], 0.0)
      o0_ref[...] = h[:, :DH]
      o1_ref[...] = h[:, DH:]

  return pl.pallas_call(
      body,
      grid=(2, NT),
      in_specs=[
          pl.BlockSpec((TR, DH), lambda p, i: (i, 0)),
          pl.BlockSpec((TR, DH), lambda p, i: (i, 0)),
          pl.BlockSpec((DH, D_HID), lambda p, i: (0, 0)),
          pl.BlockSpec((DH, D_HID), lambda p, i: (0, 0)),
          pl.BlockSpec((1, D_HID), lambda p, i: (0, 0)),
          pl.BlockSpec((1, D_HID), lambda p, i: (0, 0)),
          pl.BlockSpec((1, D_HID), lambda p, i: (0, 0)),
      ],
      out_specs=(
          pl.BlockSpec((TR, DH), lambda p, i: (i, 0)),
          pl.BlockSpec((TR, DH), lambda p, i: (i, 0)),
      ),
      out_shape=(
          jax.ShapeDtypeStruct((N_NODES, DH), jnp.float32),
          jax.ShapeDtypeStruct((N_NODES, DH), jnp.float32),
      ),
      scratch_shapes=[
          pltpu.VMEM((2, D_HID), jnp.float32),
          pltpu.VMEM((2, D_HID), jnp.float32),
      ],
  )(h0, h1, W1a, W1b, b1, gamma, beta)


def _tc_mlp2(a0, a1, W2a, W2b, b2):
  def body(a0_ref, a1_ref, w2a_ref, w2b_ref, b2_ref, o_ref):
    o_ref[...] = (
        jnp.dot(a0_ref[...], w2a_ref[...], preferred_element_type=jnp.float32)
        + jnp.dot(a1_ref[...], w2b_ref[...], preferred_element_type=jnp.float32)
        + b2_ref[...])

  return pl.pallas_call(
      body,
      grid=(NT,),
      in_specs=[
          pl.BlockSpec((TR, DH), lambda i: (i, 0)),
          pl.BlockSpec((TR, DH), lambda i: (i, 0)),
          pl.BlockSpec((DH, D_OUT), lambda i: (0, 0)),
          pl.BlockSpec((DH, D_OUT), lambda i: (0, 0)),
          pl.BlockSpec((1, D_OUT), lambda i: (0, 0)),
      ],
      out_specs=pl.BlockSpec((TR, D_OUT), lambda i: (i, 0)),
      out_shape=jax.ShapeDtypeStruct((N_NODES, D_OUT), jnp.float32),
  )(a0, a1, W2a, W2b, b2)


def kernel(x, edge_index, W1, b1, gamma, beta, W2, b2):
  x = x.astype(jnp.float32)
  pad = EPT_PAD - EPT
  src3 = jnp.pad(
      edge_index[0].astype(jnp.int32).reshape(N_SUB, EPT), ((0, 0), (0, pad)),
      constant_values=0).reshape(N_SUB, NST, CPS, EB)
  dst3 = jnp.pad(
      edge_index[1].astype(jnp.int32).reshape(N_SUB, EPT), ((0, 0), (0, pad)),
      constant_values=DUMP).reshape(N_SUB, NST, CPS, EB)
  x0 = x[:, :DH]
  x1 = x[:, DH:]
  h0, h1 = _sc_agg(x0, x1, src3, dst3)
  g0, g1 = _tc_mlp_bn_relu(
      h0, h1, W1[:DH], W1[DH:], b1.reshape(1, -1),
      gamma.reshape(1, -1), beta.reshape(1, -1))
  a0, a1 = _sc_agg(g0, g1, src3, dst3)
  return _tc_mlp2(a0, a1, W2[:DH], W2[DH:], b2.reshape(1, -1))


# speedup vs baseline: 2.0768x; 2.0768x over previous
import functools

import jax
import jax.numpy as jnp
from jax import lax
from jax.experimental import pallas as pl
from jax.experimental.pallas import tpu as pltpu
from jax.experimental.pallas import tpu_sc as plsc

N_NODES = 10000
N_EDGES = 160000
D_IN = 256
D_HID = 256
D_OUT = 128

DH = 128
N_SUB = 16
EPT = N_EDGES // N_SUB
EB = 80
NCH = EPT // EB
NST = 5
CPS = NCH // NST
RA = 624
RB = N_NODES - 15 * RA

TR = 1000
NT = N_NODES // TR


def _copy_rows(src_ref, dst_ref, s):
  @pl.when(s < 15)
  def _():
    r0 = pl.multiple_of(s * RA, 8)
    pltpu.sync_copy(src_ref.at[pl.ds(r0, RA)], dst_ref.at[pl.ds(r0, RA)])

  @pl.when(s == 15)
  def _():
    pltpu.sync_copy(src_ref.at[pl.ds(15 * RA, RB)],
                    dst_ref.at[pl.ds(15 * RA, RB)])


def _sc_agg(x0, x1, src3, dst3):
  mesh = plsc.VectorSubcoreMesh(core_axis_name="c", subcore_axis_name="s")

  @functools.partial(
      pl.kernel,
      out_type=(
          jax.ShapeDtypeStruct((N_NODES, DH), jnp.float32),
          jax.ShapeDtypeStruct((N_NODES, DH), jnp.float32),
      ),
      mesh=mesh,
      scratch_types=[
          pltpu.VMEM_SHARED((N_NODES, DH), jnp.float32),
          pltpu.VMEM((CPS, EB), jnp.int32),
          pltpu.VMEM((CPS, EB), jnp.int32),
          pltpu.VMEM((EB, DH), jnp.float32),
          pltpu.VMEM((EB, DH), jnp.float32),
          pltpu.VMEM((EB, DH), jnp.float32),
          pltpu.VMEM((EB, DH), jnp.float32),
          pltpu.SemaphoreType.DMA,
          pltpu.SemaphoreType.DMA,
          pltpu.SemaphoreType.DMA,
          pltpu.SemaphoreType.DMA,
          pltpu.SemaphoreType.DMA,
          pltpu.SemaphoreType.DMA,
      ],
  )
  def agg_kernel(x0_hbm, x1_hbm, src_hbm, dst_hbm, out0_hbm, out1_hbm,
                 acc, src_v, dst_v, rows_a, rows_b, rows_c, rows_d,
                 sem_a, sem_b, sem_c, sem_d, sem_sa, sem_sb):
    c = lax.axis_index("c")
    s = lax.axis_index("s")

    def do_half(x_hbm, out_hbm):
      _copy_rows(x_hbm, acc, s)
      plsc.subcore_barrier()

      @pl.loop(0, NST)
      def _(t):
        ci = pltpu.async_copy(src_hbm.at[s, t], src_v, sem_sa)
        cj = pltpu.async_copy(dst_hbm.at[s, t], dst_v, sem_sb)
        ci.wait()
        cj.wait()

        pltpu.async_copy(x_hbm.at[src_v.at[0]], rows_a, sem_a)
        pltpu.async_copy(x_hbm.at[src_v.at[1]], rows_b, sem_b)
        pltpu.async_copy(x_hbm.at[src_v.at[2]], rows_c, sem_c)

        @pl.loop(0, CPS - 1, step=4)
        def _(j):
          pltpu.async_copy(x_hbm.at[src_v.at[j + 3]], rows_d, sem_d)
          pltpu.make_async_copy(x_hbm.at[src_v.at[j]], rows_a, sem_a).wait()
          pltpu.sync_copy(rows_a, acc.at[dst_v.at[j]], add=True)
          pltpu.async_copy(x_hbm.at[src_v.at[j + 4]], rows_a, sem_a)
          pltpu.make_async_copy(x_hbm.at[src_v.at[j + 1]], rows_b, sem_b).wait()
          pltpu.sync_copy(rows_b, acc.at[dst_v.at[j + 1]], add=True)

          @pl.when(j + 5 < CPS)
          def _():
            pltpu.async_copy(x_hbm.at[src_v.at[j + 5]], rows_b, sem_b)

          pltpu.make_async_copy(x_hbm.at[src_v.at[j + 2]], rows_c, sem_c).wait()
          pltpu.sync_copy(rows_c, acc.at[dst_v.at[j + 2]], add=True)

          @pl.when(j + 6 < CPS)
          def _():
            pltpu.async_copy(x_hbm.at[src_v.at[j + 6]], rows_c, sem_c)

          pltpu.make_async_copy(x_hbm.at[src_v.at[j + 3]], rows_d, sem_d).wait()
          pltpu.sync_copy(rows_d, acc.at[dst_v.at[j + 3]], add=True)

        pltpu.make_async_copy(x_hbm.at[src_v.at[CPS - 1]], rows_a, sem_a).wait()
        pltpu.sync_copy(rows_a, acc.at[dst_v.at[CPS - 1]], add=True)

      plsc.subcore_barrier()
      _copy_rows(acc, out_hbm, s)

    @pl.when(c == 0)
    def _():
      do_half(x0_hbm, out0_hbm)

    @pl.when(c == 1)
    def _():
      do_half(x1_hbm, out1_hbm)

  return agg_kernel(x0, x1, src3, dst3)


def _tc_mlp_bn_relu(h0, h1, W1a, W1b, b1, gamma, beta):
  def body(h0_ref, h1_ref, w1a_ref, w1b_ref, b1_ref, g_ref, bt_ref,
           o0_ref, o1_ref, acc_ref, ss_ref):
    p = pl.program_id(0)
    i = pl.program_id(1)
    y = (jnp.dot(h0_ref[...], w1a_ref[...], preferred_element_type=jnp.float32)
         + jnp.dot(h1_ref[...], w1b_ref[...], preferred_element_type=jnp.float32)
         + b1_ref[...])

    @pl.when(jnp.logical_and(p == 0, i == 0))
    def _():
      acc_ref[...] = jnp.zeros_like(acc_ref)

    @pl.when(p == 0)
    def _():
      acc_ref[0:1, :] += jnp.sum(y, axis=0, keepdims=True)
      acc_ref[1:2, :] += jnp.sum(y * y, axis=0, keepdims=True)

    @pl.when(p == 1)
    def _():
      @pl.when(i == 0)
      def _():
        mean = acc_ref[0:1, :] * (1.0 / N_NODES)
        var = acc_ref[1:2, :] * (1.0 / N_NODES) - mean * mean
        rstd = lax.rsqrt(var + 1e-5)
        ss_ref[0:1, :] = g_ref[...] * rstd
        ss_ref[1:2, :] = bt_ref[...] - g_ref[...] * rstd * mean

      h = jnp.maximum(y * ss_ref[0:1, :] + ss_ref[1:2, :], 0.0)
      o0_ref[...] = h[:, :DH]
      o1_ref[...] = h[:, DH:]

  return pl.pallas_call(
      body,
      grid=(2, NT),
      in_specs=[
          pl.BlockSpec((TR, DH), lambda p, i: (i, 0)),
          pl.BlockSpec((TR, DH), lambda p, i: (i, 0)),
          pl.BlockSpec((DH, D_HID), lambda p, i: (0, 0)),
          pl.BlockSpec((DH, D_HID), lambda p, i: (0, 0)),
          pl.BlockSpec((1, D_HID), lambda p, i: (0, 0)),
          pl.BlockSpec((1, D_HID), lambda p, i: (0, 0)),
          pl.BlockSpec((1, D_HID), lambda p, i: (0, 0)),
      ],
      out_specs=(
          pl.BlockSpec((TR, DH), lambda p, i: (i, 0)),
          pl.BlockSpec((TR, DH), lambda p, i: (i, 0)),
      ),
      out_shape=(
          jax.ShapeDtypeStruct((N_NODES, DH), jnp.float32),
          jax.ShapeDtypeStruct((N_NODES, DH), jnp.float32),
      ),
      scratch_shapes=[
          pltpu.VMEM((2, D_HID), jnp.float32),
          pltpu.VMEM((2, D_HID), jnp.float32),
      ],
  )(h0, h1, W1a, W1b, b1, gamma, beta)


def _tc_mlp2(a0, a1, W2a, W2b, b2):
  def body(a0_ref, a1_ref, w2a_ref, w2b_ref, b2_ref, o_ref):
    o_ref[...] = (
        jnp.dot(a0_ref[...], w2a_ref[...], preferred_element_type=jnp.float32)
        + jnp.dot(a1_ref[...], w2b_ref[...], preferred_element_type=jnp.float32)
        + b2_ref[...])

  return pl.pallas_call(
      body,
      grid=(NT,),
      in_specs=[
          pl.BlockSpec((TR, DH), lambda i: (i, 0)),
          pl.BlockSpec((TR, DH), lambda i: (i, 0)),
          pl.BlockSpec((DH, D_OUT), lambda i: (0, 0)),
          pl.BlockSpec((DH, D_OUT), lambda i: (0, 0)),
          pl.BlockSpec((1, D_OUT), lambda i: (0, 0)),
      ],
      out_specs=pl.BlockSpec((TR, D_OUT), lambda i: (i, 0)),
      out_shape=jax.ShapeDtypeStruct((N_NODES, D_OUT), jnp.float32),
  )(a0, a1, W2a, W2b, b2)


def kernel(x, edge_index, W1, b1, gamma, beta, W2, b2):
  x = x.astype(jnp.float32)
  src3 = edge_index[0].astype(jnp.int32).reshape(N_SUB, NST, CPS, EB)
  dst3 = edge_index[1].astype(jnp.int32).reshape(N_SUB, NST, CPS, EB)
  x0 = x[:, :DH]
  x1 = x[:, DH:]
  h0, h1 = _sc_agg(x0, x1, src3, dst3)
  g0, g1 = _tc_mlp_bn_relu(
      h0, h1, W1[:DH], W1[DH:], b1.reshape(1, -1),
      gamma.reshape(1, -1), beta.reshape(1, -1))
  a0, a1 = _sc_agg(g0, g1, src3, dst3)
  return _tc_mlp2(a0, a1, W2[:DH], W2[DH:], b2.reshape(1, -1))
